# baseline (speedup 1.0000x reference)
import functools
import math

import jax
import jax.numpy as jnp
from jax.experimental import pallas as pl
from jax.experimental.pallas import tpu as pltpu

_N = 32768
_E = 64
_K = 2
_CF = 1.25
_MIN_CAP = 4
_B = 1024
_NB = _N // _B


def _capacity(n, e):
    cap = math.floor(_K * _CF * n / e)
    cap += cap % 2
    return max(cap, _MIN_CAP)


_CAP = float(_capacity(_N, _E))
_EPS = float(jnp.finfo(jnp.float32).eps)


def _top2(x):
    m = jnp.max(x, axis=1, keepdims=True)
    e = jnp.exp(x - m)
    z = jnp.sum(e, axis=1, keepdims=True)
    probs = e / z
    iota = jax.lax.broadcasted_iota(jnp.int32, x.shape, 1)
    big = jnp.int32(2**30)
    idx1 = jnp.min(jnp.where(x == m, iota, big), axis=1, keepdims=True)
    mask1 = iota == idx1
    x2 = jnp.where(mask1, -jnp.inf, x)
    m2 = jnp.max(x2, axis=1, keepdims=True)
    idx2 = jnp.min(jnp.where(x2 == m2, iota, big), axis=1, keepdims=True)
    mask2 = iota == idx2
    return m, z, probs, mask1, mask2, m2


def _stats_body(x_ref, h1_ref, h2_ref, aux_ref, z_ref,
                me_acc, h1_acc, h2_acc, z_acc):
    i = pl.program_id(0)
    x = x_ref[...]
    m, z, probs, mask1, mask2, _ = _top2(x)
    h1 = jnp.sum(mask1.astype(jnp.float32), axis=0, keepdims=True)
    h2 = jnp.sum(mask2.astype(jnp.float32), axis=0, keepdims=True)
    h1_ref[...] = h1[None]
    h2_ref[...] = h2[None]
    me = jnp.sum(probs, axis=0, keepdims=True)
    logz = m + jnp.log(z)
    zsq = jnp.sum(logz * logz)

    @pl.when(i == 0)
    def _():
        me_acc[...] = jnp.zeros_like(me_acc)
        h1_acc[...] = jnp.zeros_like(h1_acc)
        h2_acc[...] = jnp.zeros_like(h2_acc)
        z_acc[0, 0] = 0.0

    me_acc[...] += me
    h1_acc[...] += h1
    h2_acc[...] += h2
    z_acc[0, 0] += zsq

    me_t = me_acc[...] / _N
    ce_t = (h1_acc[...] + h2_acc[...]) / (2.0 * _N)
    aux_ref[0, 0] = _E * jnp.sum(me_t * ce_t)
    z_ref[0, 0] = z_acc[0, 0] / _N


def _cumsum0(m):
    y = m
    d = 1
    while d < m.shape[0]:
        y = y + jnp.concatenate(
            [jnp.zeros((d, m.shape[1]), y.dtype), y[: m.shape[0] - d]], axis=0)
        d *= 2
    return y


def _combine_body(x_ref, h1_ref, h2_ref, out_ref):
    i = pl.program_id(0)
    x = x_ref[...]
    m, z, _, mask1, mask2, m2 = _top2(x)
    p1 = 1.0 / z
    p2 = jnp.exp(m2 - m) / z

    h1 = h1_ref[...][:, 0, :]
    h2 = h2_ref[...][:, 0, :]
    rows = jax.lax.broadcasted_iota(jnp.int32, (_NB, _E), 0)
    before = rows < i
    base1 = jnp.sum(jnp.where(before, h1, 0.0), axis=0, keepdims=True)
    tot1 = jnp.sum(h1, axis=0, keepdims=True)
    base2 = jnp.sum(jnp.where(before, h2, 0.0), axis=0, keepdims=True) + tot1

    m1f = mask1.astype(jnp.float32)
    m2f = mask2.astype(jnp.float32)
    rank1 = base1 + _cumsum0(m1f) - 1.0
    rank2 = base2 + _cumsum0(m2f) - 1.0
    m1k = m1f * (rank1 < _CAP).astype(jnp.float32)
    m2k = m2f * (rank2 < _CAP).astype(jnp.float32)

    w1 = p1 * jnp.sum(m1k, axis=1, keepdims=True)
    w2 = p2 * jnp.sum(m2k, axis=1, keepdims=True)
    den = jnp.maximum(w1 + w2, _EPS)
    out_ref[...] = (w1 / den) * m1k + (w2 / den) * m2k


@jax.jit
def kernel(inputs):
    n, e = inputs.shape
    h1, h2, aux, zl = pl.pallas_call(
        _stats_body,
        grid=(_NB,),
        in_specs=[pl.BlockSpec((_B, _E), lambda i: (i, 0))],
        out_specs=[
            pl.BlockSpec((1, 1, _E), lambda i: (i, 0, 0)),
            pl.BlockSpec((1, 1, _E), lambda i: (i, 0, 0)),
            pl.BlockSpec(memory_space=pltpu.SMEM),
            pl.BlockSpec(memory_space=pltpu.SMEM),
        ],
        out_shape=[
            jax.ShapeDtypeStruct((_NB, 1, _E), jnp.float32),
            jax.ShapeDtypeStruct((_NB, 1, _E), jnp.float32),
            jax.ShapeDtypeStruct((1, 1), jnp.float32),
            jax.ShapeDtypeStruct((1, 1), jnp.float32),
        ],
        scratch_shapes=[
            pltpu.VMEM((1, _E), jnp.float32),
            pltpu.VMEM((1, _E), jnp.float32),
            pltpu.VMEM((1, _E), jnp.float32),
            pltpu.SMEM((1, 1), jnp.float32),
        ],
    )(inputs)

    combine = pl.pallas_call(
        _combine_body,
        grid=(_NB,),
        in_specs=[
            pl.BlockSpec((_B, _E), lambda i: (i, 0)),
            pl.BlockSpec((_NB, 1, _E), lambda i: (0, 0, 0)),
            pl.BlockSpec((_NB, 1, _E), lambda i: (0, 0, 0)),
        ],
        out_specs=pl.BlockSpec((_B, _E), lambda i: (i, 0)),
        out_shape=jax.ShapeDtypeStruct((n, e), jnp.float32),
    )(inputs, h1, h2)

    return combine, aux[0, 0], zl[0, 0]
